# baseline (speedup 1.0000x reference)
import functools

import jax
import jax.numpy as jnp
from jax import lax
from jax.experimental import pallas as pl
from jax.experimental.pallas import tpu as pltpu
from jax.experimental.pallas import tpu_sc as plsc

_NC, _NS, _L = 2, 16, 16
_NW = _NC * _NS
_C = 32
_G = 8


def kernel(x, pe_table):
    B, S, D = x.shape
    rows_per_w = S // _NW
    nspans = rows_per_w // _C
    nvec = D // _L
    mesh = plsc.VectorSubcoreMesh(core_axis_name="c", subcore_axis_name="s")

    @functools.partial(
        pl.kernel,
        out_type=jax.ShapeDtypeStruct((B, S, D), x.dtype),
        mesh=mesh,
        scratch_types=[
            pltpu.VMEM((_C, D), jnp.float32),
            pltpu.VMEM((_C, D), jnp.float32),
            pltpu.VMEM((_C, D), jnp.float32),
            pltpu.SemaphoreType.DMA,
            pltpu.SemaphoreType.DMA,
            pltpu.SemaphoreType.DMA,
            pltpu.SemaphoreType.DMA,
            pltpu.SemaphoreType.DMA,
        ],
    )
    def sc_k(x_hbm, pe_hbm, out_hbm, xb0, xb1, pb, sx0, sx1, spe, so0, so1):
        xb = (xb0, xb1)
        sx = (sx0, sx1)
        so = (so0, so1)

        cid = lax.axis_index("c")
        sid = lax.axis_index("s")
        wid = sid * _NC + cid
        seq0 = wid * rows_per_w

        def add_span(xbuf):
            def row_body(r, carry):
                for g in range(nvec // _G):
                    pvs = [
                        pb[r, pl.ds((g * _G + k) * _L, _L)]
                        for k in range(_G)
                    ]
                    for k in range(_G):
                        plsc.addupdate(
                            xbuf.at[r, pl.ds((g * _G + k) * _L, _L)], pvs[k]
                        )
                return carry

            lax.fori_loop(0, _C, row_body, 0)

        def pe_rows(j):
            return pe_hbm.at[pl.ds(seq0 + j * _C, _C)]

        def x_rows(j, b):
            return x_hbm.at[b, pl.ds(seq0 + j * _C, _C)]

        def out_rows(j, b):
            return out_hbm.at[b, pl.ds(seq0 + j * _C, _C)]

        pltpu.async_copy(pe_rows(0), pb, spe)
        pltpu.async_copy(x_rows(0, 0), xb[0], sx[0])

        def window(w, carry):
            sx_d = [None, None]
            so_d = [None, None]
            for b in range(B):
                q = b % 2
                qn = 1 - q
                if b == 0:
                    pltpu.make_async_copy(pe_rows(w), pb, spe).wait()
                if b == 0:
                    @pl.when(w > 0)
                    def _():
                        pltpu.make_async_copy(
                            xb[qn], out_rows(w, 1), so[qn]
                        ).wait()
                    sx_d[qn] = pltpu.async_copy(x_rows(w, 1), xb[qn], sx[qn])
                elif b < B - 1:
                    so_d[qn].wait()
                    sx_d[qn] = pltpu.async_copy(
                        x_rows(w, b + 1), xb[qn], sx[qn]
                    )
                else:
                    so_d[qn].wait()

                    @pl.when(w + 1 < nspans)
                    def _():
                        sx_d[qn] = pltpu.async_copy(
                            x_rows(w + 1, 0), xb[qn], sx[qn]
                        )
                if sx_d[q] is not None:
                    sx_d[q].wait()
                else:
                    pltpu.make_async_copy(x_rows(w, b), xb[q], sx[q]).wait()
                add_span(xb[q])
                if b == B - 1:
                    @pl.when(w + 1 < nspans)
                    def _():
                        pltpu.async_copy(pe_rows(w + 1), pb, spe)
                so_d[q] = pltpu.async_copy(xb[q], out_rows(w, b), so[q])
            return carry

        lax.fori_loop(0, nspans, window, 0)
        pltpu.make_async_copy(
            xb[(B - 1) % 2], out_rows(nspans - 1, B - 1), so[(B - 1) % 2]
        ).wait()

    return sc_k(x, pe_table)


# speedup vs baseline: 1.0809x; 1.0809x over previous
import functools

import jax
import jax.numpy as jnp
from jax import lax
from jax.experimental import pallas as pl
from jax.experimental.pallas import tpu as pltpu
from jax.experimental.pallas import tpu_sc as plsc

_NC, _NS, _L = 2, 16, 16
_NW = _NC * _NS
_C = 16
_G = 8
_NXB = 4


def kernel(x, pe_table):
    B, S, D = x.shape
    rows_per_w = S // _NW
    nchunks = rows_per_w // _C
    nvec = D // _L
    nsteps_win = 2 * B
    mesh = plsc.VectorSubcoreMesh(core_axis_name="c", subcore_axis_name="s")

    @functools.partial(
        pl.kernel,
        out_type=jax.ShapeDtypeStruct((B, S, D), x.dtype),
        mesh=mesh,
        scratch_types=(
            [pltpu.VMEM((_C, D), jnp.float32) for _ in range(_NXB + 2)]
            + [pltpu.SemaphoreType.DMA for _ in range(2 * _NXB + 2)]
        ),
    )
    def sc_k(x_hbm, pe_hbm, out_hbm, *bufs_and_sems):
        xb = bufs_and_sems[:_NXB]
        pb = bufs_and_sems[_NXB:_NXB + 2]
        sx = bufs_and_sems[_NXB + 2:2 * _NXB + 2]
        spe = bufs_and_sems[2 * _NXB + 2:2 * _NXB + 4]
        so = bufs_and_sems[2 * _NXB + 4:3 * _NXB + 4]

        cid = lax.axis_index("c")
        sid = lax.axis_index("s")
        wid = sid * _NC + cid
        seq0 = wid * rows_per_w

        def add_chunk(pbuf, xbuf):
            def row_body(rr, carry):
                for u in range(2):
                    r = 2 * rr + u
                    for g in range(nvec // _G):
                        pvs = [
                            pbuf[r, pl.ds((g * _G + k) * _L, _L)]
                            for k in range(_G)
                        ]
                        for k in range(_G):
                            plsc.addupdate(
                                xbuf.at[r, pl.ds((g * _G + k) * _L, _L)],
                                pvs[k],
                            )
                return carry

            lax.fori_loop(0, _C // 2, row_body, 0)

        def pe_rows(j):
            return pe_hbm.at[pl.ds(seq0 + j * _C, _C)]

        def x_rows(j, b):
            return x_hbm.at[b, pl.ds(seq0 + j * _C, _C)]

        def out_rows(j, b):
            return out_hbm.at[b, pl.ds(seq0 + j * _C, _C)]

        pltpu.async_copy(pe_rows(0), pb[0], spe[0])
        pltpu.async_copy(x_rows(0, 0), xb[0], sx[0])

        def window(jj, carry):
            sx_d = [None] * _NXB
            so_d = [None] * _NXB
            for t in range(nsteps_win):
                pj, b = divmod(t, B)
                j = jj + pj
                q = t % _NXB
                qn = (t + 1) % _NXB
                if t == 0:
                    pltpu.async_copy(pe_rows(jj + 1), pb[1], spe[1])
                    pltpu.make_async_copy(pe_rows(j), pb[0], spe[0]).wait()
                elif t == B:
                    @pl.when(jj + 2 < nchunks)
                    def _():
                        pltpu.async_copy(pe_rows(jj + 2), pb[0], spe[0])
                    pltpu.make_async_copy(pe_rows(j), pb[1], spe[1]).wait()
                jn, bn = divmod(t + 1, B)
                if t >= 3:
                    so_d[qn].wait()
                else:
                    @pl.when(jj > 0)
                    def _():
                        pltpu.make_async_copy(
                            xb[qn], out_rows(j, bn), so[qn]
                        ).wait()
                if t < nsteps_win - 1:
                    sx_d[qn] = pltpu.async_copy(
                        x_rows(jj + jn, bn), xb[qn], sx[qn]
                    )
                else:
                    @pl.when(jj + 2 < nchunks)
                    def _():
                        sx_d[qn] = pltpu.async_copy(
                            x_rows(jj + 2, 0), xb[qn], sx[qn]
                        )
                if sx_d[q] is not None:
                    sx_d[q].wait()
                else:
                    pltpu.make_async_copy(x_rows(j, b), xb[q], sx[q]).wait()
                add_chunk(pb[pj], xb[q])
                so_d[q] = pltpu.async_copy(xb[q], out_rows(j, b), so[q])
            return carry

        lax.fori_loop(0, nchunks // 2, lambda w, c: window(2 * w, c), 0)
        for t in range(nsteps_win - 3, nsteps_win):
            pltpu.make_async_copy(
                xb[t % _NXB], out_rows(nchunks - 1, t - B), so[t % _NXB]
            ).wait()

    return sc_k(x, pe_table)


# speedup vs baseline: 1.0824x; 1.0014x over previous
import functools

import jax
import jax.numpy as jnp
from jax import lax
from jax.experimental import pallas as pl
from jax.experimental.pallas import tpu as pltpu
from jax.experimental.pallas import tpu_sc as plsc

_NC, _NS, _L = 2, 16, 16
_NW = _NC * _NS
_C = 16
_G = 8
_NXB = 4


def kernel(x, pe_table):
    B, S, D = x.shape
    rows_per_w = S // _NW
    nchunks = rows_per_w // _C
    nvec = D // _L
    nsteps_win = 2 * B
    mesh = plsc.VectorSubcoreMesh(core_axis_name="c", subcore_axis_name="s")

    @functools.partial(
        pl.kernel,
        out_type=jax.ShapeDtypeStruct((B, S, D), x.dtype),
        mesh=mesh,
        scratch_types=(
            [pltpu.VMEM((_C, D), jnp.float32) for _ in range(_NXB + 2)]
            + [pltpu.SemaphoreType.DMA for _ in range(2 * _NXB + 2)]
        ),
    )
    def sc_k(x_hbm, pe_hbm, out_hbm, *bufs_and_sems):
        xb = bufs_and_sems[:_NXB]
        pb = bufs_and_sems[_NXB:_NXB + 2]
        sx = bufs_and_sems[_NXB + 2:2 * _NXB + 2]
        spe = bufs_and_sems[2 * _NXB + 2:2 * _NXB + 4]
        so = bufs_and_sems[2 * _NXB + 4:3 * _NXB + 4]

        cid = lax.axis_index("c")
        sid = lax.axis_index("s")
        wid = sid * _NC + cid
        seq0 = wid * rows_per_w

        def add_chunk(pbuf, xbuf):
            def row_body(r, carry):
                for g in range(nvec // _G):
                    pvs = [
                        pbuf[r, pl.ds((g * _G + k) * _L, _L)]
                        for k in range(_G)
                    ]
                    for k in range(_G):
                        plsc.addupdate(
                            xbuf.at[r, pl.ds((g * _G + k) * _L, _L)], pvs[k]
                        )
                return carry

            lax.fori_loop(0, _C, row_body, 0)

        def pe_rows(j):
            return pe_hbm.at[pl.ds(seq0 + j * _C, _C)]

        def x_rows(j, b):
            return x_hbm.at[b, pl.ds(seq0 + j * _C, _C)]

        def out_rows(j, b):
            return out_hbm.at[b, pl.ds(seq0 + j * _C, _C)]

        pltpu.async_copy(pe_rows(0), pb[0], spe[0])
        pltpu.async_copy(x_rows(0, 0), xb[0], sx[0])

        def window(jj, carry):
            sx_d = [None] * _NXB
            so_d = [None] * _NXB
            for t in range(nsteps_win):
                pj, b = divmod(t, B)
                j = jj + pj
                q = t % _NXB
                qn = (t + 1) % _NXB
                if t == 0:
                    pltpu.async_copy(pe_rows(jj + 1), pb[1], spe[1])
                    pltpu.make_async_copy(pe_rows(j), pb[0], spe[0]).wait()
                elif t == B:
                    @pl.when(jj + 2 < nchunks)
                    def _():
                        pltpu.async_copy(pe_rows(jj + 2), pb[0], spe[0])
                    pltpu.make_async_copy(pe_rows(j), pb[1], spe[1]).wait()
                jn, bn = divmod(t + 1, B)
                if t >= 3:
                    so_d[qn].wait()
                else:
                    @pl.when(jj > 0)
                    def _():
                        pltpu.make_async_copy(
                            xb[qn], out_rows(j, bn), so[qn]
                        ).wait()
                if t < nsteps_win - 1:
                    sx_d[qn] = pltpu.async_copy(
                        x_rows(jj + jn, bn), xb[qn], sx[qn]
                    )
                else:
                    @pl.when(jj + 2 < nchunks)
                    def _():
                        sx_d[qn] = pltpu.async_copy(
                            x_rows(jj + 2, 0), xb[qn], sx[qn]
                        )
                if sx_d[q] is not None:
                    sx_d[q].wait()
                else:
                    pltpu.make_async_copy(x_rows(j, b), xb[q], sx[q]).wait()
                add_chunk(pb[pj], xb[q])
                so_d[q] = pltpu.async_copy(xb[q], out_rows(j, b), so[q])
            return carry

        lax.fori_loop(0, nchunks // 2, lambda w, c: window(2 * w, c), 0)
        for t in range(nsteps_win - 3, nsteps_win):
            pltpu.make_async_copy(
                xb[t % _NXB], out_rows(nchunks - 1, t - B), so[t % _NXB]
            ).wait()

    return sc_k(x, pe_table)
